# baseline (speedup 1.0000x reference)
import jax
import jax.numpy as jnp
from jax.experimental import pallas as pl
from jax.experimental.pallas import tpu as pltpu


def _body(x_ref, c_ref, o_ref):
    x = x_ref[0]
    m = jnp.max(x, axis=-1, keepdims=True)
    e = jnp.exp(x - m)
    s = jnp.sum(e, axis=-1, keepdims=True)
    proj = jnp.dot(e, c_ref[...], preferred_element_type=jnp.float32)
    o_ref[0] = proj / s


def kernel(base_logits, compatibility_matrix):
    b, seq, vocab = base_logits.shape
    diac = compatibility_matrix.shape[1]

    out = pl.pallas_call(
        _body,
        grid=(b,),
        in_specs=[
            pl.BlockSpec((1, seq, vocab), lambda i: (i, 0, 0)),
            pl.BlockSpec((vocab, diac), lambda i: (0, 0)),
        ],
        out_specs=pl.BlockSpec((1, seq, diac), lambda i: (i, 0, 0)),
        out_shape=jax.ShapeDtypeStruct((b, seq, diac), jnp.float32),
        compiler_params=pltpu.CompilerParams(
            dimension_semantics=("parallel",),
        ),
    )(base_logits, compatibility_matrix)
    return out


# speedup vs baseline: 1.1915x; 1.1915x over previous
import jax
import jax.numpy as jnp
from jax.experimental import pallas as pl
from jax.experimental.pallas import tpu as pltpu


_BB = 4


def _body(x_ref, c_ref, o_ref):
    x = x_ref[...].reshape(-1, x_ref.shape[-1])
    m = jnp.max(x, axis=-1, keepdims=True)
    e = jnp.exp(x - m)
    s = jnp.sum(e, axis=-1, keepdims=True)
    proj = jnp.dot(e, c_ref[...], preferred_element_type=jnp.float32)
    o_ref[...] = (proj / s).reshape(o_ref.shape)


def kernel(base_logits, compatibility_matrix):
    b, seq, vocab = base_logits.shape
    diac = compatibility_matrix.shape[1]

    out = pl.pallas_call(
        _body,
        grid=(b // _BB,),
        in_specs=[
            pl.BlockSpec((_BB, seq, vocab), lambda i: (i, 0, 0)),
            pl.BlockSpec((vocab, diac), lambda i: (0, 0)),
        ],
        out_specs=pl.BlockSpec((_BB, seq, diac), lambda i: (i, 0, 0)),
        out_shape=jax.ShapeDtypeStruct((b, seq, diac), jnp.float32),
        compiler_params=pltpu.CompilerParams(
            dimension_semantics=("parallel",),
        ),
    )(base_logits, compatibility_matrix)
    return out


# speedup vs baseline: 3.2016x; 2.6870x over previous
import jax
import jax.numpy as jnp
from jax.experimental import pallas as pl
from jax.experimental.pallas import tpu as pltpu


def _body(x_ref, c_ref, o_ref):
    x = x_ref[0]
    m = jnp.max(x, axis=0, keepdims=True)
    e = jnp.exp(x - m)
    s = jnp.sum(e, axis=0, keepdims=True)
    proj = jax.lax.dot_general(
        c_ref[...], e, (((0,), (0,)), ((), ())),
        preferred_element_type=jnp.float32,
    )
    o_ref[0] = proj * (1.0 / s)


def kernel(base_logits, compatibility_matrix):
    b, seq, vocab = base_logits.shape
    diac = compatibility_matrix.shape[1]

    xt = jnp.transpose(base_logits, (0, 2, 1))
    out_t = pl.pallas_call(
        _body,
        grid=(b,),
        in_specs=[
            pl.BlockSpec((1, vocab, seq), lambda i: (i, 0, 0)),
            pl.BlockSpec((vocab, diac), lambda i: (0, 0)),
        ],
        out_specs=pl.BlockSpec((1, diac, seq), lambda i: (i, 0, 0)),
        out_shape=jax.ShapeDtypeStruct((b, diac, seq), jnp.float32),
        compiler_params=pltpu.CompilerParams(
            dimension_semantics=("parallel",),
        ),
    )(xt, compatibility_matrix)
    return jnp.transpose(out_t, (0, 2, 1))


# speedup vs baseline: 3.2026x; 1.0003x over previous
import jax
import jax.numpy as jnp
from jax.experimental import pallas as pl
from jax.experimental.pallas import tpu as pltpu


def _body(x0_ref, x1_ref, c_ref, o_ref):
    half = x0_ref.shape[-1]
    for k, xr in enumerate((x0_ref, x1_ref)):
        x = xr[0]
        m = jnp.max(x, axis=0, keepdims=True)
        e = jnp.exp(x - m)
        s = jnp.sum(e, axis=0, keepdims=True)
        proj = jax.lax.dot_general(
            c_ref[...], e, (((0,), (0,)), ((), ())),
            preferred_element_type=jnp.float32,
        )
        o_ref[0, :, k * half:(k + 1) * half] = proj * (1.0 / s)


def kernel(base_logits, compatibility_matrix):
    b, seq, vocab = base_logits.shape
    diac = compatibility_matrix.shape[1]

    xt = jnp.transpose(base_logits, (0, 2, 1))
    out_t = pl.pallas_call(
        _body,
        grid=(b,),
        in_specs=[
            pl.BlockSpec((1, vocab, seq // 2), lambda i: (i, 0, 0)),
            pl.BlockSpec((1, vocab, seq // 2), lambda i: (i, 0, 1)),
            pl.BlockSpec((vocab, diac), lambda i: (0, 0)),
        ],
        out_specs=pl.BlockSpec((1, diac, seq), lambda i: (i, 0, 0)),
        out_shape=jax.ShapeDtypeStruct((b, diac, seq), jnp.float32),
        compiler_params=pltpu.CompilerParams(
            dimension_semantics=("parallel",),
        ),
    )(xt, xt, compatibility_matrix)
    return jnp.transpose(out_t, (0, 2, 1))


# speedup vs baseline: 3.2545x; 1.0162x over previous
import jax
import jax.numpy as jnp
from jax.experimental import pallas as pl
from jax.experimental.pallas import tpu as pltpu


def _body(x_ref, c_ref, o_ref):
    x = x_ref[0]
    e = jnp.exp(x)
    proj = jax.lax.dot_general(
        c_ref[...], e, (((0,), (0,)), ((), ())),
        preferred_element_type=jnp.float32,
    )
    d = o_ref.shape[1]
    o_ref[0] = proj[:d] * (1.0 / proj[d:d + 1])


def kernel(base_logits, compatibility_matrix):
    b, seq, vocab = base_logits.shape
    diac = compatibility_matrix.shape[1]

    xt = jnp.transpose(base_logits, (0, 2, 1))
    caug = jnp.concatenate(
        [compatibility_matrix, jnp.ones((vocab, 1), jnp.float32)], axis=1
    )
    out_t = pl.pallas_call(
        _body,
        grid=(b,),
        in_specs=[
            pl.BlockSpec((1, vocab, seq), lambda i: (i, 0, 0)),
            pl.BlockSpec((vocab, diac + 1), lambda i: (0, 0)),
        ],
        out_specs=pl.BlockSpec((1, diac, seq), lambda i: (i, 0, 0)),
        out_shape=jax.ShapeDtypeStruct((b, diac, seq), jnp.float32),
        compiler_params=pltpu.CompilerParams(
            dimension_semantics=("parallel",),
        ),
    )(xt, caug)
    return jnp.transpose(out_t, (0, 2, 1))


# speedup vs baseline: 7.1147x; 2.1861x over previous
import jax
import jax.numpy as jnp
from jax.experimental import pallas as pl
from jax.experimental.pallas import tpu as pltpu

_BB = 8


def _body(x_ref, c_ref, o_ref):
    d = o_ref.shape[1]
    for bb in range(x_ref.shape[0]):
        x = x_ref[bb]
        e = jnp.exp(x)
        proj = jax.lax.dot_general(
            c_ref[...], e, (((0,), (0,)), ((), ())),
            preferred_element_type=jnp.float32,
        )
        o_ref[bb] = proj[:d] * (1.0 / proj[d:d + 1])


def kernel(base_logits, compatibility_matrix):
    b, seq, vocab = base_logits.shape
    diac = compatibility_matrix.shape[1]

    xt = jnp.transpose(base_logits, (0, 2, 1))
    caug = jnp.concatenate(
        [compatibility_matrix, jnp.ones((vocab, 1), jnp.float32)], axis=1
    )
    out_t = pl.pallas_call(
        _body,
        grid=(b // _BB,),
        in_specs=[
            pl.BlockSpec((_BB, vocab, seq), lambda i: (i, 0, 0)),
            pl.BlockSpec((vocab, diac + 1), lambda i: (0, 0)),
        ],
        out_specs=pl.BlockSpec((_BB, diac, seq), lambda i: (i, 0, 0)),
        out_shape=jax.ShapeDtypeStruct((b, diac, seq), jnp.float32),
        compiler_params=pltpu.CompilerParams(
            dimension_semantics=("parallel",),
        ),
    )(xt, caug)
    return jnp.transpose(out_t, (0, 2, 1))


# speedup vs baseline: 7.2744x; 1.0224x over previous
import jax
import jax.numpy as jnp
from jax.experimental import pallas as pl
from jax.experimental.pallas import tpu as pltpu

_BB = 16


def _body(x_ref, c_ref, o_ref):
    d = o_ref.shape[1]
    for bb in range(x_ref.shape[0]):
        x = x_ref[bb]
        e = jnp.exp(x)
        proj = jax.lax.dot_general(
            c_ref[...], e, (((0,), (0,)), ((), ())),
            preferred_element_type=jnp.float32,
        )
        o_ref[bb] = proj[:d] * (1.0 / proj[d:d + 1])


def kernel(base_logits, compatibility_matrix):
    b, seq, vocab = base_logits.shape
    diac = compatibility_matrix.shape[1]

    xt = jnp.transpose(base_logits, (0, 2, 1))
    caug = jnp.concatenate(
        [compatibility_matrix, jnp.ones((vocab, 1), jnp.float32)], axis=1
    )
    out_t = pl.pallas_call(
        _body,
        grid=(b // _BB,),
        in_specs=[
            pl.BlockSpec((_BB, vocab, seq), lambda i: (i, 0, 0)),
            pl.BlockSpec((vocab, diac + 1), lambda i: (0, 0)),
        ],
        out_specs=pl.BlockSpec((_BB, diac, seq), lambda i: (i, 0, 0)),
        out_shape=jax.ShapeDtypeStruct((b, diac, seq), jnp.float32),
        compiler_params=pltpu.CompilerParams(
            dimension_semantics=("parallel",),
        ),
    )(xt, caug)
    return jnp.transpose(out_t, (0, 2, 1))
